# Initial kernel scaffold; baseline (speedup 1.0000x reference)
import jax
import jax.numpy as jnp
from jax.experimental import pallas as pl


def kernel(x, token_mask, W, b):
    raise NotImplementedError("write your pallas kernel here")


# speedup vs baseline: 18.1740x; 18.1740x over previous
import functools

import jax
import jax.numpy as jnp
from jax.experimental import pallas as pl


_K = 64
_ITERS = 26


def _encoder_body(n_sae_tiles, n_chunks, x_ref, w_ref, b_ref, o_ref):
    br, d_model = x_ref.shape
    d_sae = o_ref.shape[1]
    tile = d_sae // n_sae_tiles
    j = pl.program_id(1)

    h = jax.lax.dot_general(
        x_ref[...], w_ref[...],
        (((1,), (1,)), ((), ())),
        preferred_element_type=jnp.float32,
        precision=jax.lax.Precision.DEFAULT,
    )
    o_ref[:, pl.ds(j * tile, tile)] = h + b_ref[...]

    @pl.when(j == n_sae_tiles - 1)
    def _select():
        cw = d_sae // n_chunks
        tiny = jnp.float32(1e-30)
        m = jnp.full((br, 1), -jnp.inf, jnp.float32)
        for c in range(n_chunks):
            m = jnp.maximum(
                m, jnp.max(o_ref[:, c * cw:(c + 1) * cw], axis=1,
                           keepdims=True))
        lo0 = jnp.full((br, 1), tiny, jnp.float32)
        hi0 = jnp.maximum(m, lo0)

        def it(_, carry):
            lo, hi = carry
            mid = 0.5 * (lo + hi)
            cnt = jnp.zeros((br, 1), jnp.float32)
            for c in range(n_chunks):
                hc = o_ref[:, c * cw:(c + 1) * cw]
                cnt = cnt + jnp.sum((hc >= mid).astype(jnp.float32),
                                    axis=1, keepdims=True)
            ge = cnt >= _K
            return (jnp.where(ge, mid, lo), jnp.where(ge, hi, mid))

        lo, _ = jax.lax.fori_loop(0, _ITERS, it, (lo0, hi0))

        for c in range(n_chunks):
            sl = slice(c * cw, (c + 1) * cw)
            hc = o_ref[:, sl]
            o_ref[:, sl] = jnp.where(hc >= lo, hc, 0.0)


def kernel(x, token_mask, W, b):
    batch, seq, d_model = x.shape
    d_sae = W.shape[0]
    n = batch * seq
    xf = x.reshape(n, d_model)

    br = min(256, n)
    sae_tile = min(2048, d_sae)
    n_sae_tiles = d_sae // sae_tile
    n_chunks = n_sae_tiles

    out = pl.pallas_call(
        functools.partial(_encoder_body, n_sae_tiles, n_chunks),
        grid=(n // br, n_sae_tiles),
        in_specs=[
            pl.BlockSpec((br, d_model), lambda i, j: (i, 0)),
            pl.BlockSpec((sae_tile, d_model), lambda i, j: (j, 0)),
            pl.BlockSpec((1, sae_tile), lambda i, j: (0, j)),
        ],
        out_specs=pl.BlockSpec((br, d_sae), lambda i, j: (i, 0)),
        out_shape=jax.ShapeDtypeStruct((n, d_sae), jnp.float32),
    )(xf, W, b.reshape(1, d_sae))
    return out.reshape(batch, seq, d_sae)


# speedup vs baseline: 21.6441x; 1.1909x over previous
import functools

import jax
import jax.numpy as jnp
from jax.experimental import pallas as pl
from jax.experimental.pallas import tpu as pltpu


_K = 64
_MAX_ITERS = 34


def _encoder_body(n_sae_tiles, n_chunks, x_ref, w_ref, b_ref, o_ref,
                  rmax_ref, npos_ref):
    br, d_model = x_ref.shape
    d_sae = o_ref.shape[1]
    tile = d_sae // n_sae_tiles
    j = pl.program_id(1)

    h = jax.lax.dot_general(
        x_ref[...], w_ref[...],
        (((1,), (1,)), ((), ())),
        preferred_element_type=jnp.float32,
        precision=jax.lax.Precision.DEFAULT,
    )
    h = h + b_ref[...]
    o_ref[:, pl.ds(j * tile, tile)] = h

    tmax = jnp.max(h, axis=1, keepdims=True)
    tpos = jnp.sum((h > 0).astype(jnp.float32), axis=1, keepdims=True)

    @pl.when(j == 0)
    def _init():
        rmax_ref[...] = tmax
        npos_ref[...] = tpos

    @pl.when(j > 0)
    def _acc():
        rmax_ref[...] = jnp.maximum(rmax_ref[...], tmax)
        npos_ref[...] = npos_ref[...] + tpos

    @pl.when(j == n_sae_tiles - 1)
    def _select():
        cw = d_sae // n_chunks
        tiny = jnp.float32(1e-30)
        lo0 = jnp.full((br, 1), tiny, jnp.float32)
        hi0 = jnp.maximum(rmax_ref[...], lo0)
        res0 = (npos_ref[...] <= _K).astype(jnp.float32)
        kf = jnp.float32(_K)

        def cond(carry):
            i, _, _, res = carry
            return jnp.logical_and(i < _MAX_ITERS, jnp.min(res) < 0.5)

        def body(carry):
            i, lo, hi, res = carry
            mid = 0.5 * (lo + hi)
            cnt = jnp.zeros((br, 1), jnp.float32)
            for c in range(n_chunks):
                hc = o_ref[:, c * cw:(c + 1) * cw]
                cnt = cnt + jnp.sum((hc >= mid).astype(jnp.float32),
                                    axis=1, keepdims=True)
            ge = cnt >= kf
            lo = jnp.where(ge, mid, lo)
            hi = jnp.where(ge, hi, mid)
            res = jnp.where(ge, (cnt == kf).astype(jnp.float32), res)
            return i + 1, lo, hi, res

        _, lo, _, _ = jax.lax.while_loop(
            cond, body, (jnp.int32(0), lo0, hi0, res0))

        for c in range(n_chunks):
            sl = slice(c * cw, (c + 1) * cw)
            hc = o_ref[:, sl]
            o_ref[:, sl] = jnp.where(hc >= lo, hc, 0.0)


def kernel(x, token_mask, W, b):
    batch, seq, d_model = x.shape
    d_sae = W.shape[0]
    n = batch * seq
    xf = x.reshape(n, d_model)

    br = min(256, n)
    sae_tile = min(2048, d_sae)
    n_sae_tiles = d_sae // sae_tile
    n_chunks = n_sae_tiles

    out = pl.pallas_call(
        functools.partial(_encoder_body, n_sae_tiles, n_chunks),
        grid=(n // br, n_sae_tiles),
        in_specs=[
            pl.BlockSpec((br, d_model), lambda i, j: (i, 0)),
            pl.BlockSpec((sae_tile, d_model), lambda i, j: (j, 0)),
            pl.BlockSpec((1, sae_tile), lambda i, j: (0, j)),
        ],
        out_specs=pl.BlockSpec((br, d_sae), lambda i, j: (i, 0)),
        out_shape=jax.ShapeDtypeStruct((n, d_sae), jnp.float32),
        scratch_shapes=[
            pltpu.VMEM((br, 1), jnp.float32),
            pltpu.VMEM((br, 1), jnp.float32),
        ],
    )(xf, W, b.reshape(1, d_sae))
    return out.reshape(batch, seq, d_sae)


# speedup vs baseline: 24.5409x; 1.1338x over previous
import functools

import jax
import jax.numpy as jnp
from jax.experimental import pallas as pl
from jax.experimental.pallas import tpu as pltpu


_K = 64
_MAX_ITERS = 34


def _matmul_body(x_ref, w_ref, b_ref, h_ref, rmax_ref, npos_ref):
    h = jax.lax.dot_general(
        x_ref[...], w_ref[...],
        (((1,), (1,)), ((), ())),
        preferred_element_type=jnp.float32,
        precision=jax.lax.Precision.DEFAULT,
    )
    h = h + b_ref[...]
    h_ref[...] = h
    rmax_ref[0, :, :] = jnp.max(h, axis=1, keepdims=True)
    npos_ref[0, :, :] = jnp.sum((h > 0).astype(jnp.float32), axis=1,
                                keepdims=True)


def _select_body(n_chunks, rmax_ref, npos_ref, h_ref, o_ref):
    br, d_sae = o_ref.shape
    cw = d_sae // n_chunks
    tiny = jnp.float32(1e-30)
    lo0 = jnp.full((br, 1), tiny, jnp.float32)
    hi0 = jnp.maximum(jnp.max(rmax_ref[...], axis=0), lo0)
    npos = jnp.sum(npos_ref[...], axis=0)
    res0 = (npos <= _K).astype(jnp.float32)
    kf = jnp.float32(_K)

    def cond(carry):
        i, _, _, res = carry
        return jnp.logical_and(i < _MAX_ITERS, jnp.min(res) < 0.5)

    def body(carry):
        i, lo, hi, res = carry
        mid = 0.5 * (lo + hi)
        cnt = jnp.zeros((br, 1), jnp.float32)
        for c in range(n_chunks):
            hc = h_ref[:, c * cw:(c + 1) * cw]
            cnt = cnt + jnp.sum((hc >= mid).astype(jnp.float32),
                                axis=1, keepdims=True)
        ge = cnt >= kf
        lo = jnp.where(ge, mid, lo)
        hi = jnp.where(ge, hi, mid)
        res = jnp.where(ge, (cnt == kf).astype(jnp.float32), res)
        return i + 1, lo, hi, res

    _, lo, _, _ = jax.lax.while_loop(
        cond, body, (jnp.int32(0), lo0, hi0, res0))

    for c in range(n_chunks):
        sl = slice(c * cw, (c + 1) * cw)
        hc = h_ref[:, sl]
        o_ref[:, sl] = jnp.where(hc >= lo, hc, 0.0)


def kernel(x, token_mask, W, b):
    batch, seq, d_model = x.shape
    d_sae = W.shape[0]
    n = batch * seq
    xf = x.reshape(n, d_model)

    br1 = min(512, n)
    sae_tile = min(2048, d_sae)
    n_sae_tiles = d_sae // sae_tile
    h, rmax_p, npos_p = pl.pallas_call(
        _matmul_body,
        grid=(n_sae_tiles, n // br1),
        in_specs=[
            pl.BlockSpec((br1, d_model), lambda j, i: (i, 0)),
            pl.BlockSpec((sae_tile, d_model), lambda j, i: (j, 0)),
            pl.BlockSpec((1, sae_tile), lambda j, i: (0, j)),
        ],
        out_specs=[
            pl.BlockSpec((br1, sae_tile), lambda j, i: (i, j)),
            pl.BlockSpec((1, br1, 1), lambda j, i: (j, i, 0)),
            pl.BlockSpec((1, br1, 1), lambda j, i: (j, i, 0)),
        ],
        out_shape=[
            jax.ShapeDtypeStruct((n, d_sae), jnp.float32),
            jax.ShapeDtypeStruct((n_sae_tiles, n, 1), jnp.float32),
            jax.ShapeDtypeStruct((n_sae_tiles, n, 1), jnp.float32),
        ],
    )(xf, W, b.reshape(1, d_sae))

    br2 = min(128, n)
    n_chunks = n_sae_tiles
    out = pl.pallas_call(
        functools.partial(_select_body, n_chunks),
        grid=(n // br2,),
        in_specs=[
            pl.BlockSpec((n_sae_tiles, br2, 1), lambda i: (0, i, 0)),
            pl.BlockSpec((n_sae_tiles, br2, 1), lambda i: (0, i, 0)),
            pl.BlockSpec((br2, d_sae), lambda i: (i, 0)),
        ],
        out_specs=pl.BlockSpec((br2, d_sae), lambda i: (i, 0)),
        out_shape=jax.ShapeDtypeStruct((n, d_sae), jnp.float32),
        input_output_aliases={2: 0},
    )(rmax_p, npos_p, h)
    return out.reshape(batch, seq, d_sae)
